# Initial kernel scaffold; baseline (speedup 1.0000x reference)
import jax
import jax.numpy as jnp
from jax.experimental import pallas as pl


def kernel(input):
    raise NotImplementedError("write your pallas kernel here")


# speedup vs baseline: 1.0359x; 1.0359x over previous
import math

import numpy as np
import jax
import jax.numpy as jnp
from jax.experimental import pallas as pl
from jax.experimental.pallas import tpu as pltpu

_FIFTY_PERCENT_STD = 0.8696735925295497


def _erfi(x):
    total = 0.0
    for k in range(40):
        total += x ** (2 * k + 1) / (math.factorial(k) * (2 * k + 1))
    return 2.0 / math.sqrt(math.pi) * total


_GOAL_STD = math.sqrt(2.0) * _erfi(1.0 - 0.05)

_B, _R, _C = 2, 8192, 2048
_ROWS = _B * _R
_N = _ROWS * _C


def _np_threefry2x32(k0, k1, x0, x1):
    def rotl(v, d):
        return ((v << np.uint32(d)) | (v >> np.uint32(32 - d))).astype(np.uint32)

    ks = [np.uint32(k0), np.uint32(k1),
          np.uint32(np.uint32(k0) ^ np.uint32(k1) ^ np.uint32(0x1BD11BDA))]
    rotations = [[13, 15, 26, 6], [17, 29, 16, 24]]
    x0 = (x0 + ks[0]).astype(np.uint32)
    x1 = (x1 + ks[1]).astype(np.uint32)
    for i in range(5):
        for r in rotations[i % 2]:
            x0 = (x0 + x1).astype(np.uint32)
            x1 = rotl(x1, r)
            x1 = (x0 ^ x1).astype(np.uint32)
        x0 = (x0 + ks[(i + 1) % 3]).astype(np.uint32)
        x1 = (x1 + ks[(i + 2) % 3] + np.uint32(i + 1)).astype(np.uint32)
    return x0, x1


_FK0, _FK1 = _np_threefry2x32(0, 0, np.uint32(0), np.uint32(1))
_FK0, _FK1 = int(_FK0), int(_FK1)


_STAT_ROWS = 1024


def _stats_body(x_ref, acc_ref):
    i = pl.program_id(0)

    @pl.when(i == 0)
    def _():
        acc_ref[0] = 0.0
        acc_ref[1] = 0.0

    x = x_ref[...]
    acc_ref[0] += jnp.sum(x)
    acc_ref[1] += jnp.sum(x * x)


_MASK_ROWS = 256

_U_LO = float(np.nextafter(np.float32(-1.0), np.float32(0.0)))
_U_SCALE = float(np.float32(1.0) - np.float32(_U_LO))
_SQRT2 = float(np.array(np.sqrt(2), np.float32))


def _mask_body(params_ref, x_ref, o_ref):
    i = pl.program_id(0)
    base = jnp.uint32(i * _MASK_ROWS * _C)
    row = jax.lax.broadcasted_iota(jnp.uint32, (_MASK_ROWS, _C), 0)
    col = jax.lax.broadcasted_iota(jnp.uint32, (_MASK_ROWS, _C), 1)
    cnt = base + row * jnp.uint32(_C) + col

    ks0 = jnp.uint32(_FK0)
    ks1 = jnp.uint32(_FK1)
    ks = [ks0, ks1, jnp.uint32(_FK0 ^ _FK1 ^ 0x1BD11BDA)]
    rotations = [[13, 15, 26, 6], [17, 29, 16, 24]]
    x0 = jnp.full((_MASK_ROWS, _C), ks0, jnp.uint32)
    x1 = cnt + ks1
    for r in range(5):
        for d in rotations[r % 2]:
            x0 = x0 + x1
            x1 = (x1 << jnp.uint32(d)) | (x1 >> jnp.uint32(32 - d))
            x1 = x0 ^ x1
        x0 = x0 + ks[(r + 1) % 3]
        x1 = x1 + ks[(r + 2) % 3] + jnp.uint32(r + 1)
    bits = x0 ^ x1

    fb = (bits >> jnp.uint32(9)) | jnp.uint32(0x3F800000)
    f = jax.lax.bitcast_convert_type(fb, jnp.float32) - jnp.float32(1.0)
    u = jnp.maximum(jnp.float32(_U_LO),
                    f * jnp.float32(_U_SCALE) + jnp.float32(_U_LO))
    noise = jnp.float32(_SQRT2) * jax.lax.erf_inv(u)

    std = params_ref[0]
    mean = params_ref[1]
    x = x_ref[...]
    normalized = jnp.abs(x / std - mean) / _FIFTY_PERCENT_STD
    renorm = noise * normalized
    o_ref[...] = jnp.where(renorm < _GOAL_STD, jnp.float32(0.0), x)


def kernel(input):
    x2d = input.reshape(_ROWS, _C)

    sums = pl.pallas_call(
        _stats_body,
        grid=(_ROWS // _STAT_ROWS,),
        in_specs=[pl.BlockSpec((_STAT_ROWS, _C), lambda i: (i, 0))],
        out_specs=pl.BlockSpec(memory_space=pltpu.SMEM),
        out_shape=jax.ShapeDtypeStruct((2,), jnp.float32),
    )(x2d)

    n = jnp.float32(_N)
    mean = sums[0] / n
    var = (sums[1] - sums[0] * mean) / (n - 1.0)
    std = jnp.sqrt(var)
    params = jnp.stack([std, mean])

    out = pl.pallas_call(
        _mask_body,
        grid=(_ROWS // _MASK_ROWS,),
        in_specs=[
            pl.BlockSpec(memory_space=pltpu.SMEM),
            pl.BlockSpec((_MASK_ROWS, _C), lambda i: (i, 0)),
        ],
        out_specs=pl.BlockSpec((_MASK_ROWS, _C), lambda i: (i, 0)),
        out_shape=jax.ShapeDtypeStruct((_ROWS, _C), jnp.float32),
    )(params, x2d)

    return out.reshape(_B, _R, _C)


# speedup vs baseline: 1.1406x; 1.1011x over previous
import functools
import math

import numpy as np
import jax
import jax.numpy as jnp
from jax import lax
from jax.experimental import pallas as pl
from jax.experimental.pallas import tpu as pltpu
from jax.experimental.pallas import tpu_sc as plsc

_FIFTY_PERCENT_STD = 0.8696735925295497


def _erfi(x):
    total = 0.0
    for k in range(40):
        total += x ** (2 * k + 1) / (math.factorial(k) * (2 * k + 1))
    return 2.0 / math.sqrt(math.pi) * total


_GOAL_STD = math.sqrt(2.0) * _erfi(1.0 - 0.05)

_B, _R, _C = 2, 8192, 2048
_ROWS = _B * _R
_N = _ROWS * _C


def _np_threefry2x32(k0, k1, x0, x1):
    def rotl(v, d):
        return ((v << np.uint32(d)) | (v >> np.uint32(32 - d))).astype(np.uint32)

    ks = [np.uint32(k0), np.uint32(k1),
          np.uint32(np.uint32(k0) ^ np.uint32(k1) ^ np.uint32(0x1BD11BDA))]
    rotations = [[13, 15, 26, 6], [17, 29, 16, 24]]
    x0 = (x0 + ks[0]).astype(np.uint32)
    x1 = (x1 + ks[1]).astype(np.uint32)
    for i in range(5):
        for r in rotations[i % 2]:
            x0 = (x0 + x1).astype(np.uint32)
            x1 = rotl(x1, r)
            x1 = (x0 ^ x1).astype(np.uint32)
        x0 = (x0 + ks[(i + 1) % 3]).astype(np.uint32)
        x1 = (x1 + ks[(i + 2) % 3] + np.uint32(i + 1)).astype(np.uint32)
    return x0, x1


_FK0, _FK1 = _np_threefry2x32(0, 0, np.uint32(0), np.uint32(1))
_FK0, _FK1 = int(_FK0), int(_FK1)

_U_LO = float(np.nextafter(np.float32(-1.0), np.float32(0.0)))
_U_SCALE = float(np.float32(1.0) - np.float32(_U_LO))
_SQRT2 = float(np.array(np.sqrt(2), np.float32))

_ROTATIONS = [[13, 15, 26, 6], [17, 29, 16, 24]]


_TAIL_ROWS = 4096
_HEAD_ROWS = _ROWS - _TAIL_ROWS
_TAIL_N = _TAIL_ROWS * _C
_TAIL_FLAT0 = _HEAD_ROWS * _C


_STAT_ROWS = 1024


def _stats_body(x_ref, acc_ref):
    i = pl.program_id(0)

    @pl.when(i == 0)
    def _():
        acc_ref[0] = 0.0
        acc_ref[1] = 0.0

    x = x_ref[...]
    acc_ref[0] += jnp.sum(x)
    acc_ref[1] += jnp.sum(x * x)


_NSUB = 32
_CH = 16384
_LANES = 16
_WAYS = 4


def _sc_tf_u(cnt_list):
    ks0 = jnp.uint32(_FK0)
    ks1 = jnp.uint32(_FK1)
    ks = [ks0, ks1, jnp.uint32(_FK0 ^ _FK1 ^ 0x1BD11BDA)]
    x0s = [jnp.full((_LANES,), ks0, jnp.uint32) for _ in cnt_list]
    x1s = [c + ks1 for c in cnt_list]
    for r in range(5):
        for d in _ROTATIONS[r % 2]:
            x0s = [a + b for a, b in zip(x0s, x1s)]
            x1s = [(b << jnp.uint32(d)) | (b >> jnp.uint32(32 - d)) for b in x1s]
            x1s = [a ^ b for a, b in zip(x0s, x1s)]
        x0s = [a + ks[(r + 1) % 3] for a in x0s]
        x1s = [b + ks[(r + 2) % 3] + jnp.uint32(r + 1) for b in x1s]
    us = []
    for a, b in zip(x0s, x1s):
        bits = a ^ b
        fb = (bits >> jnp.uint32(9)) | jnp.uint32(0x3F800000)
        f = lax.bitcast_convert_type(fb, jnp.float32) - jnp.float32(1.0)
        us.append(jnp.maximum(jnp.float32(_U_LO),
                              f * jnp.float32(_U_SCALE) + jnp.float32(_U_LO)))
    return us


def _make_sc_ugen(total_n, flat0):
    assert total_n % (_NSUB * _CH) == 0
    per_sub = total_n // _NSUB
    n_chunks = per_sub // _CH
    mesh = plsc.VectorSubcoreMesh(core_axis_name="c", subcore_axis_name="s")

    @functools.partial(
        pl.kernel, mesh=mesh,
        out_type=jax.ShapeDtypeStruct((total_n,), jnp.float32),
        scratch_types=[pltpu.VMEM((_CH,), jnp.float32)],
    )
    def ugen(out_hbm, buf):
        wid = lax.axis_index("s") * 2 + lax.axis_index("c")
        sub_base = flat0 + wid * per_sub

        def chunk_body(c, carry):
            chunk_base = sub_base + c * _CH

            def inner(i, carry2):
                base = chunk_base + i * (_LANES * _WAYS)
                iota = lax.iota(jnp.int32, _LANES)
                cnts = [(base + w * _LANES + iota).astype(jnp.uint32)
                        for w in range(_WAYS)]
                us = _sc_tf_u(cnts)
                off = i * (_LANES * _WAYS)
                for w in range(_WAYS):
                    buf[pl.ds(off + w * _LANES, _LANES)] = us[w]
                return carry2

            lax.fori_loop(0, _CH // (_LANES * _WAYS), inner, 0, unroll=2)
            pltpu.sync_copy(buf, out_hbm.at[pl.ds(wid * per_sub + c * _CH, _CH)])
            return carry

        lax.fori_loop(0, n_chunks, chunk_body, 0)

    return ugen


_sc_ugen = _make_sc_ugen(_TAIL_N, _TAIL_FLAT0)


_MASK_ROWS = 256


def _noise_mask(u, params_ref, x):
    noise = jnp.float32(_SQRT2) * lax.erf_inv(u)
    std = params_ref[0]
    mean = params_ref[1]
    normalized = jnp.abs(x / std - mean) / _FIFTY_PERCENT_STD
    renorm = noise * normalized
    return jnp.where(renorm < _GOAL_STD, jnp.float32(0.0), x)


def _mask_head_body(params_ref, x_ref, o_ref):
    i = pl.program_id(0)
    base = jnp.uint32(i * _MASK_ROWS * _C)
    row = lax.broadcasted_iota(jnp.uint32, (_MASK_ROWS, _C), 0)
    col = lax.broadcasted_iota(jnp.uint32, (_MASK_ROWS, _C), 1)
    cnt = base + row * jnp.uint32(_C) + col

    ks0 = jnp.uint32(_FK0)
    ks1 = jnp.uint32(_FK1)
    ks = [ks0, ks1, jnp.uint32(_FK0 ^ _FK1 ^ 0x1BD11BDA)]
    x0 = jnp.full((_MASK_ROWS, _C), ks0, jnp.uint32)
    x1 = cnt + ks1
    for r in range(5):
        for d in _ROTATIONS[r % 2]:
            x0 = x0 + x1
            x1 = (x1 << jnp.uint32(d)) | (x1 >> jnp.uint32(32 - d))
            x1 = x0 ^ x1
        x0 = x0 + ks[(r + 1) % 3]
        x1 = x1 + ks[(r + 2) % 3] + jnp.uint32(r + 1)
    bits = x0 ^ x1

    fb = (bits >> jnp.uint32(9)) | jnp.uint32(0x3F800000)
    f = lax.bitcast_convert_type(fb, jnp.float32) - jnp.float32(1.0)
    u = jnp.maximum(jnp.float32(_U_LO),
                    f * jnp.float32(_U_SCALE) + jnp.float32(_U_LO))
    o_ref[...] = _noise_mask(u, params_ref, x_ref[...])


def _mask_tail_body(params_ref, x_ref, u_ref, prev_ref, o_ref):
    del prev_ref
    o_ref[...] = _noise_mask(u_ref[...], params_ref, x_ref[...])


def kernel(input):
    x2d = input.reshape(_ROWS, _C)

    u_tail = _sc_ugen().reshape(_TAIL_ROWS, _C)

    sums = pl.pallas_call(
        _stats_body,
        grid=(_ROWS // _STAT_ROWS,),
        in_specs=[pl.BlockSpec((_STAT_ROWS, _C), lambda i: (i, 0))],
        out_specs=pl.BlockSpec(memory_space=pltpu.SMEM),
        out_shape=jax.ShapeDtypeStruct((2,), jnp.float32),
    )(x2d)

    n = jnp.float32(_N)
    mean = sums[0] / n
    var = (sums[1] - sums[0] * mean) / (n - 1.0)
    std = jnp.sqrt(var)
    params = jnp.stack([std, mean])

    out_head = pl.pallas_call(
        _mask_head_body,
        grid=(_HEAD_ROWS // _MASK_ROWS,),
        in_specs=[
            pl.BlockSpec(memory_space=pltpu.SMEM),
            pl.BlockSpec((_MASK_ROWS, _C), lambda i: (i, 0)),
        ],
        out_specs=pl.BlockSpec((_MASK_ROWS, _C), lambda i: (i, 0)),
        out_shape=jax.ShapeDtypeStruct((_ROWS, _C), jnp.float32),
    )(params, x2d)

    head_blocks = _HEAD_ROWS // _MASK_ROWS
    out = pl.pallas_call(
        _mask_tail_body,
        grid=(_TAIL_ROWS // _MASK_ROWS,),
        in_specs=[
            pl.BlockSpec(memory_space=pltpu.SMEM),
            pl.BlockSpec((_MASK_ROWS, _C), lambda i: (head_blocks + i, 0)),
            pl.BlockSpec((_MASK_ROWS, _C), lambda i: (i, 0)),
            pl.BlockSpec(memory_space=pl.ANY),
        ],
        out_specs=pl.BlockSpec((_MASK_ROWS, _C), lambda i: (head_blocks + i, 0)),
        out_shape=jax.ShapeDtypeStruct((_ROWS, _C), jnp.float32),
        input_output_aliases={3: 0},
    )(params, x2d, u_tail, out_head)

    return out.reshape(_B, _R, _C)


# speedup vs baseline: 1.2084x; 1.0594x over previous
import functools
import math

import numpy as np
import jax
import jax.numpy as jnp
from jax import lax
from jax.experimental import pallas as pl
from jax.experimental.pallas import tpu as pltpu
from jax.experimental.pallas import tpu_sc as plsc

_FIFTY_PERCENT_STD = 0.8696735925295497


def _erfi(x):
    total = 0.0
    for k in range(40):
        total += x ** (2 * k + 1) / (math.factorial(k) * (2 * k + 1))
    return 2.0 / math.sqrt(math.pi) * total


_GOAL_STD = math.sqrt(2.0) * _erfi(1.0 - 0.05)

_B, _R, _C = 2, 8192, 2048
_ROWS = _B * _R
_N = _ROWS * _C


def _np_threefry2x32(k0, k1, x0, x1):
    def rotl(v, d):
        return ((v << np.uint32(d)) | (v >> np.uint32(32 - d))).astype(np.uint32)

    ks = [np.uint32(k0), np.uint32(k1),
          np.uint32(np.uint32(k0) ^ np.uint32(k1) ^ np.uint32(0x1BD11BDA))]
    rotations = [[13, 15, 26, 6], [17, 29, 16, 24]]
    x0 = (x0 + ks[0]).astype(np.uint32)
    x1 = (x1 + ks[1]).astype(np.uint32)
    for i in range(5):
        for r in rotations[i % 2]:
            x0 = (x0 + x1).astype(np.uint32)
            x1 = rotl(x1, r)
            x1 = (x0 ^ x1).astype(np.uint32)
        x0 = (x0 + ks[(i + 1) % 3]).astype(np.uint32)
        x1 = (x1 + ks[(i + 2) % 3] + np.uint32(i + 1)).astype(np.uint32)
    return x0, x1


_FK0, _FK1 = _np_threefry2x32(0, 0, np.uint32(0), np.uint32(1))
_FK0, _FK1 = int(_FK0), int(_FK1)

_U_LO = float(np.nextafter(np.float32(-1.0), np.float32(0.0)))
_U_SCALE = float(np.float32(1.0) - np.float32(_U_LO))
_SQRT2 = float(np.array(np.sqrt(2), np.float32))

_ROTATIONS = [[13, 15, 26, 6], [17, 29, 16, 24]]


_TAIL_ROWS = 5888
_HEAD_ROWS = _ROWS - _TAIL_ROWS
_TAIL_N = _TAIL_ROWS * _C
_TAIL_FLAT0 = _HEAD_ROWS * _C


_STAT_ROWS = 1024


def _stats_body(x_ref, acc_ref):
    i = pl.program_id(0)

    @pl.when(i == 0)
    def _():
        acc_ref[0] = 0.0
        acc_ref[1] = 0.0

    x = x_ref[...]
    acc_ref[0] += jnp.sum(x)
    acc_ref[1] += jnp.sum(x * x)


_NSUB = 32
_CH = 16384
_LANES = 16
_WAYS = 4


def _sc_tf_u(cnt_list):
    ks0 = jnp.uint32(_FK0)
    ks1 = jnp.uint32(_FK1)
    ks = [ks0, ks1, jnp.uint32(_FK0 ^ _FK1 ^ 0x1BD11BDA)]
    x0s = [jnp.full((_LANES,), ks0, jnp.uint32) for _ in cnt_list]
    x1s = [c + ks1 for c in cnt_list]
    for r in range(5):
        for d in _ROTATIONS[r % 2]:
            x0s = [a + b for a, b in zip(x0s, x1s)]
            x1s = [(b << jnp.uint32(d)) | (b >> jnp.uint32(32 - d)) for b in x1s]
            x1s = [a ^ b for a, b in zip(x0s, x1s)]
        x0s = [a + ks[(r + 1) % 3] for a in x0s]
        x1s = [b + ks[(r + 2) % 3] + jnp.uint32(r + 1) for b in x1s]
    us = []
    for a, b in zip(x0s, x1s):
        bits = a ^ b
        fb = (bits >> jnp.uint32(9)) | jnp.uint32(0x3F800000)
        f = lax.bitcast_convert_type(fb, jnp.float32) - jnp.float32(1.0)
        us.append(jnp.maximum(jnp.float32(_U_LO),
                              f * jnp.float32(_U_SCALE) + jnp.float32(_U_LO)))
    return us


def _make_sc_ugen(total_n, flat0):
    assert total_n % (_NSUB * _CH) == 0
    per_sub = total_n // _NSUB
    n_chunks = per_sub // _CH
    mesh = plsc.VectorSubcoreMesh(core_axis_name="c", subcore_axis_name="s")

    @functools.partial(
        pl.kernel, mesh=mesh,
        out_type=jax.ShapeDtypeStruct((total_n,), jnp.float32),
        scratch_types=[pltpu.VMEM((_CH,), jnp.float32)],
    )
    def ugen(out_hbm, buf):
        wid = lax.axis_index("s") * 2 + lax.axis_index("c")
        sub_base = flat0 + wid * per_sub

        def chunk_body(c, carry):
            chunk_base = sub_base + c * _CH

            def inner(i, carry2):
                base = chunk_base + i * (_LANES * _WAYS)
                iota = lax.iota(jnp.int32, _LANES)
                cnts = [(base + w * _LANES + iota).astype(jnp.uint32)
                        for w in range(_WAYS)]
                us = _sc_tf_u(cnts)
                off = i * (_LANES * _WAYS)
                for w in range(_WAYS):
                    buf[pl.ds(off + w * _LANES, _LANES)] = us[w]
                return carry2

            lax.fori_loop(0, _CH // (_LANES * _WAYS), inner, 0, unroll=2)
            pltpu.sync_copy(buf, out_hbm.at[pl.ds(wid * per_sub + c * _CH, _CH)])
            return carry

        lax.fori_loop(0, n_chunks, chunk_body, 0)

    return ugen


_sc_ugen = _make_sc_ugen(_TAIL_N, _TAIL_FLAT0)


_MASK_ROWS = 256


def _noise_mask(u, params_ref, x):
    noise = jnp.float32(_SQRT2) * lax.erf_inv(u)
    std = params_ref[0]
    mean = params_ref[1]
    normalized = jnp.abs(x / std - mean) / _FIFTY_PERCENT_STD
    renorm = noise * normalized
    return jnp.where(renorm < _GOAL_STD, jnp.float32(0.0), x)


def _mask_head_body(params_ref, x_ref, o_ref):
    i = pl.program_id(0)
    base = jnp.uint32(i * _MASK_ROWS * _C)
    row = lax.broadcasted_iota(jnp.uint32, (_MASK_ROWS, _C), 0)
    col = lax.broadcasted_iota(jnp.uint32, (_MASK_ROWS, _C), 1)
    cnt = base + row * jnp.uint32(_C) + col

    ks0 = jnp.uint32(_FK0)
    ks1 = jnp.uint32(_FK1)
    ks = [ks0, ks1, jnp.uint32(_FK0 ^ _FK1 ^ 0x1BD11BDA)]
    x0 = jnp.full((_MASK_ROWS, _C), ks0, jnp.uint32)
    x1 = cnt + ks1
    for r in range(5):
        for d in _ROTATIONS[r % 2]:
            x0 = x0 + x1
            x1 = (x1 << jnp.uint32(d)) | (x1 >> jnp.uint32(32 - d))
            x1 = x0 ^ x1
        x0 = x0 + ks[(r + 1) % 3]
        x1 = x1 + ks[(r + 2) % 3] + jnp.uint32(r + 1)
    bits = x0 ^ x1

    fb = (bits >> jnp.uint32(9)) | jnp.uint32(0x3F800000)
    f = lax.bitcast_convert_type(fb, jnp.float32) - jnp.float32(1.0)
    u = jnp.maximum(jnp.float32(_U_LO),
                    f * jnp.float32(_U_SCALE) + jnp.float32(_U_LO))
    o_ref[...] = _noise_mask(u, params_ref, x_ref[...])


def _mask_tail_body(params_ref, x_ref, u_ref, prev_ref, o_ref):
    del prev_ref
    o_ref[...] = _noise_mask(u_ref[...], params_ref, x_ref[...])


def kernel(input):
    x2d = input.reshape(_ROWS, _C)

    u_tail = _sc_ugen().reshape(_TAIL_ROWS, _C)

    sums = pl.pallas_call(
        _stats_body,
        grid=(_ROWS // _STAT_ROWS,),
        in_specs=[pl.BlockSpec((_STAT_ROWS, _C), lambda i: (i, 0))],
        out_specs=pl.BlockSpec(memory_space=pltpu.SMEM),
        out_shape=jax.ShapeDtypeStruct((2,), jnp.float32),
    )(x2d)

    n = jnp.float32(_N)
    mean = sums[0] / n
    var = (sums[1] - sums[0] * mean) / (n - 1.0)
    std = jnp.sqrt(var)
    params = jnp.stack([std, mean])

    out_head = pl.pallas_call(
        _mask_head_body,
        grid=(_HEAD_ROWS // _MASK_ROWS,),
        in_specs=[
            pl.BlockSpec(memory_space=pltpu.SMEM),
            pl.BlockSpec((_MASK_ROWS, _C), lambda i: (i, 0)),
        ],
        out_specs=pl.BlockSpec((_MASK_ROWS, _C), lambda i: (i, 0)),
        out_shape=jax.ShapeDtypeStruct((_ROWS, _C), jnp.float32),
    )(params, x2d)

    head_blocks = _HEAD_ROWS // _MASK_ROWS
    out = pl.pallas_call(
        _mask_tail_body,
        grid=(_TAIL_ROWS // _MASK_ROWS,),
        in_specs=[
            pl.BlockSpec(memory_space=pltpu.SMEM),
            pl.BlockSpec((_MASK_ROWS, _C), lambda i: (head_blocks + i, 0)),
            pl.BlockSpec((_MASK_ROWS, _C), lambda i: (i, 0)),
            pl.BlockSpec(memory_space=pl.ANY),
        ],
        out_specs=pl.BlockSpec((_MASK_ROWS, _C), lambda i: (head_blocks + i, 0)),
        out_shape=jax.ShapeDtypeStruct((_ROWS, _C), jnp.float32),
        input_output_aliases={3: 0},
    )(params, x2d, u_tail, out_head)

    return out.reshape(_B, _R, _C)


# speedup vs baseline: 1.2160x; 1.0063x over previous
import functools
import math

import numpy as np
import jax
import jax.numpy as jnp
from jax import lax
from jax.experimental import pallas as pl
from jax.experimental.pallas import tpu as pltpu
from jax.experimental.pallas import tpu_sc as plsc

_FIFTY_PERCENT_STD = 0.8696735925295497


def _erfi(x):
    total = 0.0
    for k in range(40):
        total += x ** (2 * k + 1) / (math.factorial(k) * (2 * k + 1))
    return 2.0 / math.sqrt(math.pi) * total


_GOAL_STD = math.sqrt(2.0) * _erfi(1.0 - 0.05)

_B, _R, _C = 2, 8192, 2048
_ROWS = _B * _R
_N = _ROWS * _C


def _np_threefry2x32(k0, k1, x0, x1):
    def rotl(v, d):
        return ((v << np.uint32(d)) | (v >> np.uint32(32 - d))).astype(np.uint32)

    ks = [np.uint32(k0), np.uint32(k1),
          np.uint32(np.uint32(k0) ^ np.uint32(k1) ^ np.uint32(0x1BD11BDA))]
    rotations = [[13, 15, 26, 6], [17, 29, 16, 24]]
    x0 = (x0 + ks[0]).astype(np.uint32)
    x1 = (x1 + ks[1]).astype(np.uint32)
    for i in range(5):
        for r in rotations[i % 2]:
            x0 = (x0 + x1).astype(np.uint32)
            x1 = rotl(x1, r)
            x1 = (x0 ^ x1).astype(np.uint32)
        x0 = (x0 + ks[(i + 1) % 3]).astype(np.uint32)
        x1 = (x1 + ks[(i + 2) % 3] + np.uint32(i + 1)).astype(np.uint32)
    return x0, x1


_FK0, _FK1 = _np_threefry2x32(0, 0, np.uint32(0), np.uint32(1))
_FK0, _FK1 = int(_FK0), int(_FK1)

_U_LO = float(np.nextafter(np.float32(-1.0), np.float32(0.0)))
_U_SCALE = float(np.float32(1.0) - np.float32(_U_LO))
_SQRT2 = float(np.array(np.sqrt(2), np.float32))

_ROTATIONS = [[13, 15, 26, 6], [17, 29, 16, 24]]


_TAIL_ROWS = 5888
_HEAD_ROWS = _ROWS - _TAIL_ROWS
_TAIL_N = _TAIL_ROWS * _C
_TAIL_FLAT0 = _HEAD_ROWS * _C


_NSUB = 32
_CH = 16384
_LANES = 16
_WAYS = 4


def _sc_tf_u(cnt_list):
    ks0 = jnp.uint32(_FK0)
    ks1 = jnp.uint32(_FK1)
    ks = [ks0, ks1, jnp.uint32(_FK0 ^ _FK1 ^ 0x1BD11BDA)]
    x0s = [jnp.full((_LANES,), ks0, jnp.uint32) for _ in cnt_list]
    x1s = [c + ks1 for c in cnt_list]
    for r in range(5):
        for d in _ROTATIONS[r % 2]:
            x0s = [a + b for a, b in zip(x0s, x1s)]
            x1s = [(b << jnp.uint32(d)) | (b >> jnp.uint32(32 - d)) for b in x1s]
            x1s = [a ^ b for a, b in zip(x0s, x1s)]
        x0s = [a + ks[(r + 1) % 3] for a in x0s]
        x1s = [b + ks[(r + 2) % 3] + jnp.uint32(r + 1) for b in x1s]
    us = []
    for a, b in zip(x0s, x1s):
        bits = a ^ b
        fb = (bits >> jnp.uint32(9)) | jnp.uint32(0x3F800000)
        f = lax.bitcast_convert_type(fb, jnp.float32) - jnp.float32(1.0)
        us.append(jnp.maximum(jnp.float32(_U_LO),
                              f * jnp.float32(_U_SCALE) + jnp.float32(_U_LO)))
    return us


def _make_sc_ugen(total_n, flat0):
    assert total_n % (_NSUB * _CH) == 0
    per_sub = total_n // _NSUB
    n_chunks = per_sub // _CH
    mesh = plsc.VectorSubcoreMesh(core_axis_name="c", subcore_axis_name="s")

    @functools.partial(
        pl.kernel, mesh=mesh,
        out_type=jax.ShapeDtypeStruct((total_n,), jnp.float32),
        scratch_types=[pltpu.VMEM((_CH,), jnp.float32)],
    )
    def ugen(out_hbm, buf):
        wid = lax.axis_index("s") * 2 + lax.axis_index("c")
        sub_base = flat0 + wid * per_sub

        def chunk_body(c, carry):
            chunk_base = sub_base + c * _CH

            def inner(i, carry2):
                base = chunk_base + i * (_LANES * _WAYS)
                iota = lax.iota(jnp.int32, _LANES)
                cnts = [(base + w * _LANES + iota).astype(jnp.uint32)
                        for w in range(_WAYS)]
                us = _sc_tf_u(cnts)
                off = i * (_LANES * _WAYS)
                for w in range(_WAYS):
                    buf[pl.ds(off + w * _LANES, _LANES)] = us[w]
                return carry2

            lax.fori_loop(0, _CH // (_LANES * _WAYS), inner, 0, unroll=2)
            pltpu.sync_copy(buf, out_hbm.at[pl.ds(wid * per_sub + c * _CH, _CH)])
            return carry

        lax.fori_loop(0, n_chunks, chunk_body, 0)

    return ugen


def _sc_ugen():
    return _make_sc_ugen(_TAIL_N, _TAIL_FLAT0)()


_STAT_ROWS = 1024
_S_STEPS = _ROWS // _STAT_ROWS
_MASK_ROWS = 256
_H_STEPS = _HEAD_ROWS // _MASK_ROWS


def _noise_mask(u, std, mean, x):
    noise = jnp.float32(_SQRT2) * lax.erf_inv(u)
    normalized = jnp.abs(x / std - mean) / _FIFTY_PERCENT_STD
    renorm = noise * normalized
    return jnp.where(renorm < _GOAL_STD, jnp.float32(0.0), x)


def _tf_u_2d(base_elem):
    row = lax.broadcasted_iota(jnp.uint32, (_MASK_ROWS, _C), 0)
    col = lax.broadcasted_iota(jnp.uint32, (_MASK_ROWS, _C), 1)
    cnt = base_elem + row * jnp.uint32(_C) + col
    ks0 = jnp.uint32(_FK0)
    ks1 = jnp.uint32(_FK1)
    ks = [ks0, ks1, jnp.uint32(_FK0 ^ _FK1 ^ 0x1BD11BDA)]
    x0 = jnp.full((_MASK_ROWS, _C), ks0, jnp.uint32)
    x1 = cnt + ks1
    for r in range(5):
        for d in _ROTATIONS[r % 2]:
            x0 = x0 + x1
            x1 = (x1 << jnp.uint32(d)) | (x1 >> jnp.uint32(32 - d))
            x1 = x0 ^ x1
        x0 = x0 + ks[(r + 1) % 3]
        x1 = x1 + ks[(r + 2) % 3] + jnp.uint32(r + 1)
    bits = x0 ^ x1
    fb = (bits >> jnp.uint32(9)) | jnp.uint32(0x3F800000)
    f = lax.bitcast_convert_type(fb, jnp.float32) - jnp.float32(1.0)
    return jnp.maximum(jnp.float32(_U_LO),
                       f * jnp.float32(_U_SCALE) + jnp.float32(_U_LO))


def _main_body(xs_ref, xm_ref, o_ref, acc_ref):
    i = pl.program_id(0)

    @pl.when(i == 0)
    def _():
        acc_ref[0] = 0.0
        acc_ref[1] = 0.0

    @pl.when(i < _S_STEPS)
    def _():
        xs = xs_ref[...]
        acc_ref[0] += jnp.sum(xs)
        acc_ref[1] += jnp.sum(xs * xs)

    @pl.when(i == _S_STEPS)
    def _():
        n = jnp.float32(_N)
        mean = acc_ref[0] / n
        var = (acc_ref[1] - acc_ref[0] * mean) / (n - 1.0)
        acc_ref[2] = jnp.sqrt(var)
        acc_ref[3] = mean

    @pl.when(i >= _S_STEPS)
    def _():
        j = i - _S_STEPS
        u = _tf_u_2d(jnp.uint32(j * _MASK_ROWS * _C))
        o_ref[...] = _noise_mask(u, acc_ref[2], acc_ref[3], xm_ref[...])


def _mask_tail_body(params_ref, x_ref, u_ref, prev_ref, o_ref):
    del prev_ref
    o_ref[...] = _noise_mask(u_ref[...], params_ref[2], params_ref[3], x_ref[...])


def kernel(input):
    x2d = input.reshape(_ROWS, _C)

    u_tail = _sc_ugen().reshape(_TAIL_ROWS, _C)

    grid = _S_STEPS + _H_STEPS
    out_head, params = pl.pallas_call(
        _main_body,
        grid=(grid,),
        in_specs=[
            pl.BlockSpec((_STAT_ROWS, _C),
                         lambda i: (jnp.minimum(i, _S_STEPS - 1), 0)),
            pl.BlockSpec((_MASK_ROWS, _C),
                         lambda i: (jnp.maximum(i - _S_STEPS, 0), 0)),
        ],
        out_specs=[
            pl.BlockSpec((_MASK_ROWS, _C),
                         lambda i: (jnp.maximum(i - _S_STEPS, 0), 0)),
            pl.BlockSpec(memory_space=pltpu.SMEM),
        ],
        out_shape=[
            jax.ShapeDtypeStruct((_ROWS, _C), jnp.float32),
            jax.ShapeDtypeStruct((4,), jnp.float32),
        ],
    )(x2d, x2d)

    head_blocks = _HEAD_ROWS // _MASK_ROWS
    out = pl.pallas_call(
        _mask_tail_body,
        grid=(_TAIL_ROWS // _MASK_ROWS,),
        in_specs=[
            pl.BlockSpec(memory_space=pltpu.SMEM),
            pl.BlockSpec((_MASK_ROWS, _C), lambda i: (head_blocks + i, 0)),
            pl.BlockSpec((_MASK_ROWS, _C), lambda i: (i, 0)),
            pl.BlockSpec(memory_space=pl.ANY),
        ],
        out_specs=pl.BlockSpec((_MASK_ROWS, _C), lambda i: (head_blocks + i, 0)),
        out_shape=jax.ShapeDtypeStruct((_ROWS, _C), jnp.float32),
        input_output_aliases={3: 0},
    )(params, x2d, u_tail, out_head)

    return out.reshape(_B, _R, _C)


# speedup vs baseline: 1.3154x; 1.0817x over previous
import functools
import math

import numpy as np
import jax
import jax.numpy as jnp
from jax import lax
from jax.experimental import pallas as pl
from jax.experimental.pallas import tpu as pltpu
from jax.experimental.pallas import tpu_sc as plsc

_FIFTY_PERCENT_STD = 0.8696735925295497


def _erfi(x):
    total = 0.0
    for k in range(40):
        total += x ** (2 * k + 1) / (math.factorial(k) * (2 * k + 1))
    return 2.0 / math.sqrt(math.pi) * total


_GOAL_STD = math.sqrt(2.0) * _erfi(1.0 - 0.05)

_B, _R, _C = 2, 8192, 2048
_ROWS = _B * _R
_N = _ROWS * _C


def _np_threefry2x32(k0, k1, x0, x1):
    def rotl(v, d):
        return ((v << np.uint32(d)) | (v >> np.uint32(32 - d))).astype(np.uint32)

    ks = [np.uint32(k0), np.uint32(k1),
          np.uint32(np.uint32(k0) ^ np.uint32(k1) ^ np.uint32(0x1BD11BDA))]
    rotations = [[13, 15, 26, 6], [17, 29, 16, 24]]
    x0 = (x0 + ks[0]).astype(np.uint32)
    x1 = (x1 + ks[1]).astype(np.uint32)
    for i in range(5):
        for r in rotations[i % 2]:
            x0 = (x0 + x1).astype(np.uint32)
            x1 = rotl(x1, r)
            x1 = (x0 ^ x1).astype(np.uint32)
        x0 = (x0 + ks[(i + 1) % 3]).astype(np.uint32)
        x1 = (x1 + ks[(i + 2) % 3] + np.uint32(i + 1)).astype(np.uint32)
    return x0, x1


_FK0, _FK1 = _np_threefry2x32(0, 0, np.uint32(0), np.uint32(1))
_FK0, _FK1 = int(_FK0), int(_FK1)

_U_LO = float(np.nextafter(np.float32(-1.0), np.float32(0.0)))
_U_SCALE = float(np.float32(1.0) - np.float32(_U_LO))
_SQRT2 = float(np.array(np.sqrt(2), np.float32))

_ROTATIONS = [[13, 15, 26, 6], [17, 29, 16, 24]]


_TAIL_ROWS = 5888
_HEAD_ROWS = _ROWS - _TAIL_ROWS
_TAIL_N = _TAIL_ROWS * _C
_TAIL_FLAT0 = _HEAD_ROWS * _C


_NSUB = 32
_CH = 16384
_LANES = 16
_WAYS = 4


def _sc_tf_u(cnt_list):
    ks0 = jnp.uint32(_FK0)
    ks1 = jnp.uint32(_FK1)
    ks = [ks0, ks1, jnp.uint32(_FK0 ^ _FK1 ^ 0x1BD11BDA)]
    x0s = [jnp.full((_LANES,), ks0, jnp.uint32) for _ in cnt_list]
    x1s = [c + ks1 for c in cnt_list]
    for r in range(5):
        for d in _ROTATIONS[r % 2]:
            x0s = [a + b for a, b in zip(x0s, x1s)]
            x1s = [(b << jnp.uint32(d)) | (b >> jnp.uint32(32 - d)) for b in x1s]
            x1s = [a ^ b for a, b in zip(x0s, x1s)]
        x0s = [a + ks[(r + 1) % 3] for a in x0s]
        x1s = [b + ks[(r + 2) % 3] + jnp.uint32(r + 1) for b in x1s]
    us = []
    for a, b in zip(x0s, x1s):
        bits = a ^ b
        fb = (bits >> jnp.uint32(9)) | jnp.uint32(0x3F800000)
        f = lax.bitcast_convert_type(fb, jnp.float32) - jnp.float32(1.0)
        us.append(jnp.maximum(jnp.float32(_U_LO),
                              f * jnp.float32(_U_SCALE) + jnp.float32(_U_LO)))
    return us


_CH_ROWS = _CH // _C


def _make_sc_ugen(tail_rows, flat0):
    total_n = tail_rows * _C
    assert total_n % (_NSUB * _CH) == 0
    per_sub = total_n // _NSUB
    sub_rows = tail_rows // _NSUB
    n_chunks = per_sub // _CH
    mesh = plsc.VectorSubcoreMesh(core_axis_name="c", subcore_axis_name="s")

    @functools.partial(
        pl.kernel, mesh=mesh,
        out_type=jax.ShapeDtypeStruct((tail_rows, _C), jnp.float32),
        scratch_types=[pltpu.VMEM((_CH_ROWS, _C), jnp.float32)],
    )
    def ugen(out_hbm, buf):
        wid = lax.axis_index("s") * 2 + lax.axis_index("c")
        sub_base = flat0 + wid * per_sub

        def chunk_body(c, carry):
            chunk_base = sub_base + c * _CH

            def inner(i, carry2):
                base = chunk_base + i * (_LANES * _WAYS)
                iota = lax.iota(jnp.int32, _LANES)
                cnts = [(base + w * _LANES + iota).astype(jnp.uint32)
                        for w in range(_WAYS)]
                us = _sc_tf_u(cnts)
                r = i // (_C // (_LANES * _WAYS))
                col = (i % (_C // (_LANES * _WAYS))) * (_LANES * _WAYS)
                for w in range(_WAYS):
                    buf[r, pl.ds(col + w * _LANES, _LANES)] = us[w]
                return carry2

            lax.fori_loop(0, _CH // (_LANES * _WAYS), inner, 0, unroll=2)
            pltpu.sync_copy(
                buf,
                out_hbm.at[pl.ds(wid * sub_rows + c * _CH_ROWS, _CH_ROWS), :])
            return carry

        lax.fori_loop(0, n_chunks, chunk_body, 0)

    return ugen


def _sc_ugen():
    return _make_sc_ugen(_TAIL_ROWS, _TAIL_FLAT0)()


_STAT_ROWS = 1024
_S_STEPS = _ROWS // _STAT_ROWS
_MASK_ROWS = 256
_H_STEPS = _HEAD_ROWS // _MASK_ROWS


def _noise_mask(u, std, mean, x):
    noise = jnp.float32(_SQRT2) * lax.erf_inv(u)
    normalized = jnp.abs(x / std - mean) / _FIFTY_PERCENT_STD
    renorm = noise * normalized
    return jnp.where(renorm < _GOAL_STD, jnp.float32(0.0), x)


def _tf_u_2d(base_elem):
    row = lax.broadcasted_iota(jnp.uint32, (_MASK_ROWS, _C), 0)
    col = lax.broadcasted_iota(jnp.uint32, (_MASK_ROWS, _C), 1)
    cnt = base_elem + row * jnp.uint32(_C) + col
    ks0 = jnp.uint32(_FK0)
    ks1 = jnp.uint32(_FK1)
    ks = [ks0, ks1, jnp.uint32(_FK0 ^ _FK1 ^ 0x1BD11BDA)]
    x0 = jnp.full((_MASK_ROWS, _C), ks0, jnp.uint32)
    x1 = cnt + ks1
    for r in range(5):
        for d in _ROTATIONS[r % 2]:
            x0 = x0 + x1
            x1 = (x1 << jnp.uint32(d)) | (x1 >> jnp.uint32(32 - d))
            x1 = x0 ^ x1
        x0 = x0 + ks[(r + 1) % 3]
        x1 = x1 + ks[(r + 2) % 3] + jnp.uint32(r + 1)
    bits = x0 ^ x1
    fb = (bits >> jnp.uint32(9)) | jnp.uint32(0x3F800000)
    f = lax.bitcast_convert_type(fb, jnp.float32) - jnp.float32(1.0)
    return jnp.maximum(jnp.float32(_U_LO),
                       f * jnp.float32(_U_SCALE) + jnp.float32(_U_LO))


def _main_body(xs_ref, xm_ref, o_ref, acc_ref):
    i = pl.program_id(0)

    @pl.when(i == 0)
    def _():
        acc_ref[0] = 0.0
        acc_ref[1] = 0.0

    @pl.when(i < _S_STEPS)
    def _():
        xs = xs_ref[...]
        acc_ref[0] += jnp.sum(xs)
        acc_ref[1] += jnp.sum(xs * xs)

    @pl.when(i == _S_STEPS)
    def _():
        n = jnp.float32(_N)
        mean = acc_ref[0] / n
        var = (acc_ref[1] - acc_ref[0] * mean) / (n - 1.0)
        acc_ref[2] = jnp.sqrt(var)
        acc_ref[3] = mean

    @pl.when(i >= _S_STEPS)
    def _():
        j = i - _S_STEPS
        u = _tf_u_2d(jnp.uint32(j * _MASK_ROWS * _C))
        o_ref[...] = _noise_mask(u, acc_ref[2], acc_ref[3], xm_ref[...])


def _mask_tail_body(params_ref, x_ref, u_ref, prev_ref, o_ref):
    del prev_ref
    o_ref[...] = _noise_mask(u_ref[...], params_ref[2], params_ref[3], x_ref[...])


def kernel(input):
    x2d = input.reshape(_ROWS, _C)

    u_tail = _sc_ugen()

    grid = _S_STEPS + _H_STEPS
    out_head, params = pl.pallas_call(
        _main_body,
        grid=(grid,),
        in_specs=[
            pl.BlockSpec((_STAT_ROWS, _C),
                         lambda i: (jnp.minimum(i, _S_STEPS - 1), 0)),
            pl.BlockSpec((_MASK_ROWS, _C),
                         lambda i: (jnp.maximum(i - _S_STEPS, 0), 0)),
        ],
        out_specs=[
            pl.BlockSpec((_MASK_ROWS, _C),
                         lambda i: (jnp.maximum(i - _S_STEPS, 0), 0)),
            pl.BlockSpec(memory_space=pltpu.SMEM),
        ],
        out_shape=[
            jax.ShapeDtypeStruct((_ROWS, _C), jnp.float32),
            jax.ShapeDtypeStruct((4,), jnp.float32),
        ],
    )(x2d, x2d)

    head_blocks = _HEAD_ROWS // _MASK_ROWS
    out = pl.pallas_call(
        _mask_tail_body,
        grid=(_TAIL_ROWS // _MASK_ROWS,),
        in_specs=[
            pl.BlockSpec(memory_space=pltpu.SMEM),
            pl.BlockSpec((_MASK_ROWS, _C), lambda i: (head_blocks + i, 0)),
            pl.BlockSpec((_MASK_ROWS, _C), lambda i: (i, 0)),
            pl.BlockSpec(memory_space=pl.ANY),
        ],
        out_specs=pl.BlockSpec((_MASK_ROWS, _C), lambda i: (head_blocks + i, 0)),
        out_shape=jax.ShapeDtypeStruct((_ROWS, _C), jnp.float32),
        input_output_aliases={3: 0},
    )(params, x2d, u_tail, out_head)

    return out.reshape(_B, _R, _C)
